# Initial kernel scaffold; baseline (speedup 1.0000x reference)
import jax
import jax.numpy as jnp
from jax.experimental import pallas as pl


def kernel(x, edge_index, W1, b1, W2, b2, W3, b3, W4, b4, Wc, bc):
    raise NotImplementedError("write your pallas kernel here")


# speedup vs baseline: 6.6564x; 6.6564x over previous
import functools

import jax
import jax.numpy as jnp
from jax import lax
from jax.experimental import pallas as pl
from jax.experimental.pallas import tpu as pltpu
from jax.experimental.pallas import tpu_sc as plsc

N = 10000
E = 320000
D = 128
H = 128
C = 16

NC = 2
NS = 16
NW = NC * NS
NPAD = 10240
ROWS_PT = NPAD // NS
EPW = E // NW
CH = 80
NCH = EPW // CH
HH = H // 2
EPT = E // NS
NCH2 = EPT // CH

_MESH = plsc.VectorSubcoreMesh(
    core_axis_name="c", subcore_axis_name="s", num_cores=NC, num_subcores=NS)
_SC_PARAMS = pltpu.CompilerParams(use_tc_tiling_on_sc=False)


def _wid():
    return lax.axis_index("s") * NC + lax.axis_index("c")


@functools.partial(
    pl.kernel,
    out_type=jax.ShapeDtypeStruct((NC, 2, NPAD), jnp.float32),
    mesh=_MESH,
    compiler_params=_SC_PARAMS,
    scratch_types=[
        pltpu.VMEM((NCH, CH), jnp.int32),
        pltpu.VMEM((NCH, CH), jnp.int32),
        pltpu.VMEM((CH,), jnp.float32),
        pltpu.VMEM((ROWS_PT,), jnp.float32),
        pltpu.VMEM_SHARED((NPAD,), jnp.float32),
        pltpu.SemaphoreType.DMA,
    ],
)
def _sc_degree(src_hbm, dst_hbm, out_hbm, sidx, didx, ones_v, zbuf,
               deg_sh, sem0):
    c = lax.axis_index("c")
    s = lax.axis_index("s")
    w = _wid()
    base = s * ROWS_PT
    one = jnp.full((16,), 1.0, jnp.float32)
    zero = jnp.zeros((16,), jnp.float32)

    def fill(t, _):
        ones_v[pl.ds(t * 16, 16)] = one
        return 0
    lax.fori_loop(0, CH // 16, fill, 0)

    def zfill(t, _):
        zbuf[pl.ds(t * 16, 16)] = zero
        return 0
    lax.fori_loop(0, ROWS_PT // 16, zfill, 0)

    pltpu.sync_copy(src_hbm.at[w], sidx)
    pltpu.sync_copy(dst_hbm.at[w], didx)

    DEPTH = 8

    def histogram(idx, which):
        pltpu.sync_copy(zbuf, deg_sh.at[pl.ds(base, ROWS_PT)])
        plsc.subcore_barrier()

        def fire(j, _):
            @pl.when(j >= DEPTH)
            def _():
                pltpu.make_async_copy(ones_v, deg_sh.at[idx.at[j - DEPTH]],
                                      sem0).wait()
            pltpu.async_copy(ones_v, deg_sh.at[idx.at[j]], sem0, add=True)
            return 0
        lax.fori_loop(0, NCH, fire, 0)

        def drain(j, _):
            pltpu.make_async_copy(ones_v, deg_sh.at[idx.at[j]], sem0).wait()
            return 0
        lax.fori_loop(NCH - DEPTH, NCH, drain, 0)
        plsc.subcore_barrier()
        pltpu.sync_copy(deg_sh.at[pl.ds(base, ROWS_PT)],
                        out_hbm.at[c, which, pl.ds(base, ROWS_PT)])
        plsc.subcore_barrier()

    histogram(sidx, 0)
    histogram(didx, 1)


@functools.partial(
    pl.kernel,
    out_type=jax.ShapeDtypeStruct((NC, NPAD, HH), jnp.float32),
    mesh=_MESH,
    compiler_params=_SC_PARAMS,
    scratch_types=[
        pltpu.VMEM((NCH2, CH), jnp.int32),
        pltpu.VMEM((NCH2, CH), jnp.int32),
        pltpu.VMEM((CH, HH), jnp.float32),
        pltpu.VMEM((CH, HH), jnp.float32),
        pltpu.VMEM_SHARED((NPAD, HH), jnp.float32),
        pltpu.SemaphoreType.DMA,
        pltpu.SemaphoreType.DMA,
        pltpu.SemaphoreType.DMA,
        pltpu.SemaphoreType.DMA,
    ],
)
def _sc_segsum(hs_hbm, src_hbm, dst_hbm, out_hbm, sidx, didx, rows0, rows1,
               acc_sh, gsem0, gsem1, ssem0, ssem1):
    c = lax.axis_index("c")
    s = lax.axis_index("s")
    base = s * ROWS_PT
    zero = jnp.zeros((16,), jnp.float32)
    hs_half = hs_hbm.at[c]

    def zstore(t, _):
        rows0[t // (HH // 16), pl.ds((t % (HH // 16)) * 16, 16)] = zero
        return 0
    lax.fori_loop(0, CH * (HH // 16), zstore, 0)

    def zcopy(k, _):
        pltpu.sync_copy(rows0, acc_sh.at[pl.ds(base + k * CH, CH)])
        return 0
    lax.fori_loop(0, ROWS_PT // CH, zcopy, 0)

    pltpu.sync_copy(src_hbm.at[s], sidx)
    pltpu.sync_copy(dst_hbm.at[s], didx)
    plsc.subcore_barrier()

    pltpu.async_copy(hs_half.at[sidx.at[0]], rows0, gsem0)
    pltpu.async_copy(hs_half.at[sidx.at[1]], rows1, gsem1)

    def body(jj, _):
        a = 2 * jj
        b = a + 1
        pltpu.make_async_copy(hs_half.at[sidx.at[a]], rows0, gsem0).wait()
        pltpu.async_copy(rows0, acc_sh.at[didx.at[a]], ssem0, add=True)
        pltpu.make_async_copy(hs_half.at[sidx.at[b]], rows1, gsem1).wait()
        pltpu.async_copy(rows1, acc_sh.at[didx.at[b]], ssem1, add=True)
        pltpu.make_async_copy(rows0, acc_sh.at[didx.at[a]], ssem0).wait()

        @pl.when(a + 2 < NCH2)
        def _():
            pltpu.async_copy(hs_half.at[sidx.at[a + 2]], rows0, gsem0)
        pltpu.make_async_copy(rows1, acc_sh.at[didx.at[b]], ssem1).wait()

        @pl.when(b + 2 < NCH2)
        def _():
            pltpu.async_copy(hs_half.at[sidx.at[b + 2]], rows1, gsem1)
        return 0
    lax.fori_loop(0, NCH2 // 2, body, 0)
    plsc.subcore_barrier()

    pltpu.sync_copy(acc_sh.at[pl.ds(base, ROWS_PT)],
                    out_hbm.at[c, pl.ds(base, ROWS_PT)])


@functools.partial(
    pl.kernel,
    out_type=(jax.ShapeDtypeStruct((E, C), jnp.float32),
              jax.ShapeDtypeStruct((E, C), jnp.float32)),
    mesh=_MESH,
    compiler_params=_SC_PARAMS,
    scratch_types=[
        pltpu.VMEM((NCH, CH), jnp.int32),
        pltpu.VMEM((NCH, CH), jnp.int32),
        pltpu.VMEM((CH, C), jnp.float32),
        pltpu.VMEM((CH, C), jnp.float32),
        pltpu.VMEM((CH, C), jnp.float32),
        pltpu.VMEM((CH, C), jnp.float32),
        pltpu.SemaphoreType.DMA,
        pltpu.SemaphoreType.DMA,
        pltpu.SemaphoreType.DMA,
        pltpu.SemaphoreType.DMA,
    ],
)
def _sc_pred(p1_hbm, p2_hbm, src_hbm, dst_hbm, g1_hbm, g2_hbm,
             sidx, didx, r1a, r2a, r1b, r2b, gsem0, gsem1, wsem0, wsem1):
    w = _wid()
    ebase = w * EPW

    pltpu.sync_copy(src_hbm.at[w], sidx)
    pltpu.sync_copy(dst_hbm.at[w], didx)

    pltpu.async_copy(p1_hbm.at[sidx.at[0]], r1a, gsem0)
    pltpu.async_copy(p2_hbm.at[didx.at[0]], r2a, gsem0)
    pltpu.async_copy(p1_hbm.at[sidx.at[1]], r1b, gsem1)
    pltpu.async_copy(p2_hbm.at[didx.at[1]], r2b, gsem1)

    def body(jj, _):
        a = 2 * jj
        b = a + 1
        pltpu.make_async_copy(p1_hbm.at[sidx.at[a]], r1a, gsem0).wait()
        pltpu.make_async_copy(p2_hbm.at[didx.at[a]], r2a, gsem0).wait()
        pltpu.async_copy(r1a, g1_hbm.at[pl.ds(ebase + a * CH, CH)], wsem0)
        pltpu.async_copy(r2a, g2_hbm.at[pl.ds(ebase + a * CH, CH)], wsem0)
        pltpu.make_async_copy(p1_hbm.at[sidx.at[b]], r1b, gsem1).wait()
        pltpu.make_async_copy(p2_hbm.at[didx.at[b]], r2b, gsem1).wait()
        pltpu.async_copy(r1b, g1_hbm.at[pl.ds(ebase + b * CH, CH)], wsem1)
        pltpu.async_copy(r2b, g2_hbm.at[pl.ds(ebase + b * CH, CH)], wsem1)
        pltpu.make_async_copy(r1a, g1_hbm.at[pl.ds(ebase + a * CH, CH)],
                              wsem0).wait()
        pltpu.make_async_copy(r2a, g2_hbm.at[pl.ds(ebase + a * CH, CH)],
                              wsem0).wait()

        @pl.when(a + 2 < NCH)
        def _():
            pltpu.async_copy(p1_hbm.at[sidx.at[a + 2]], r1a, gsem0)
            pltpu.async_copy(p2_hbm.at[didx.at[a + 2]], r2a, gsem0)
        pltpu.make_async_copy(r1b, g1_hbm.at[pl.ds(ebase + b * CH, CH)],
                              wsem1).wait()
        pltpu.make_async_copy(r2b, g2_hbm.at[pl.ds(ebase + b * CH, CH)],
                              wsem1).wait()

        @pl.when(b + 2 < NCH)
        def _():
            pltpu.async_copy(p1_hbm.at[sidx.at[b + 2]], r1b, gsem1)
            pltpu.async_copy(p2_hbm.at[didx.at[b + 2]], r2b, gsem1)
        return 0
    lax.fori_loop(0, NCH // 2, body, 0)

    j = NCH - 1
    pltpu.make_async_copy(p1_hbm.at[sidx.at[j]], r1a, gsem0).wait()
    pltpu.make_async_copy(p2_hbm.at[didx.at[j]], r2a, gsem0).wait()
    pltpu.sync_copy(r1a, g1_hbm.at[pl.ds(ebase + j * CH, CH)])
    pltpu.sync_copy(r2a, g2_hbm.at[pl.ds(ebase + j * CH, CH)])


def _tc_prep(degT, x, W1):
    def body(d_ref, x_ref, w_ref, ns_ref, nd_ref, hs_ref):
        d = d_ref[...]
        out_deg = d[:, 0:1] + d[:, 2:3]
        in_deg = d[:, 1:2] + d[:, 3:4]
        ns = lax.rsqrt(jnp.maximum(out_deg, 1.0))
        nd = lax.rsqrt(jnp.maximum(in_deg, 1.0))
        ns_ref[...] = ns
        nd_ref[...] = nd
        r = jnp.dot(x_ref[...] * ns, w_ref[...],
                    preferred_element_type=jnp.float32)
        hs_ref[0, :, :] = r[:, :HH]
        hs_ref[1, :, :] = r[:, HH:]
    return pl.pallas_call(
        body,
        out_shape=(jax.ShapeDtypeStruct((NPAD, 1), jnp.float32),
                   jax.ShapeDtypeStruct((NPAD, 1), jnp.float32),
                   jax.ShapeDtypeStruct((NC, NPAD, HH), jnp.float32)),
    )(degT, x, W1)


def _tc_layer(agg, nd, ns, b, Wn):
    def body(a_ref, nd_ref, ns_ref, b_ref, w_ref, out_ref):
        nd_v = nd_ref[...]
        h_lo = jnp.maximum(a_ref[0, :, :] * nd_v + b_ref[:, :HH], 0.0)
        h_hi = jnp.maximum(a_ref[1, :, :] * nd_v + b_ref[:, HH:], 0.0)
        ns_v = ns_ref[...]
        r = (jnp.dot(h_lo * ns_v, w_ref[:HH, :],
                     preferred_element_type=jnp.float32)
             + jnp.dot(h_hi * ns_v, w_ref[HH:, :],
                       preferred_element_type=jnp.float32))
        out_ref[0, :, :] = r[:, :HH]
        out_ref[1, :, :] = r[:, HH:]
    return pl.pallas_call(
        body,
        out_shape=jax.ShapeDtypeStruct((NC, NPAD, HH), jnp.float32),
    )(agg, nd, ns, b, Wn)


def _tc_final(agg, nd, b, WcA, WcB, bc):
    def body(a_ref, nd_ref, b_ref, wa_ref, wb_ref, bc_ref, p1_ref, p2_ref):
        nd_v = nd_ref[...]
        h_lo = jnp.maximum(a_ref[0, :, :] * nd_v + b_ref[:, :HH], 0.0)
        h_hi = jnp.maximum(a_ref[1, :, :] * nd_v + b_ref[:, HH:], 0.0)
        p1_ref[...] = (jnp.dot(h_lo, wa_ref[:HH, :],
                               preferred_element_type=jnp.float32)
                       + jnp.dot(h_hi, wa_ref[HH:, :],
                                 preferred_element_type=jnp.float32)
                       + bc_ref[...])
        p2_ref[...] = (jnp.dot(h_lo, wb_ref[:HH, :],
                               preferred_element_type=jnp.float32)
                       + jnp.dot(h_hi, wb_ref[HH:, :],
                                 preferred_element_type=jnp.float32))
    return pl.pallas_call(
        body,
        out_shape=(jax.ShapeDtypeStruct((NPAD, C), jnp.float32),
                   jax.ShapeDtypeStruct((NPAD, C), jnp.float32)),
    )(agg, nd, b, WcA, WcB, bc)


_E2 = E * C // 128
_EB = _E2 // 8


def _tc_sigmoid(g1, g2):
    def body(g1_ref, g2_ref, out_ref):
        z = g1_ref[...] + g2_ref[...]
        out_ref[...] = 1.0 / (1.0 + jnp.exp(-z))
    return pl.pallas_call(
        body,
        grid=(8,),
        in_specs=[pl.BlockSpec((_EB, 128), lambda i: (i, 0)),
                  pl.BlockSpec((_EB, 128), lambda i: (i, 0))],
        out_specs=pl.BlockSpec((_EB, 128), lambda i: (i, 0)),
        out_shape=jax.ShapeDtypeStruct((_E2, 128), jnp.float32),
    )(g1, g2)


def kernel(x, edge_index, W1, b1, W2, b2, W3, b3, W4, b4, Wc, bc):
    src3 = edge_index[0].reshape(NW, NCH, CH)
    dst3 = edge_index[1].reshape(NW, NCH, CH)
    src16 = edge_index[0].reshape(NS, NCH2, CH)
    dst16 = edge_index[1].reshape(NS, NCH2, CH)
    x_pad = jnp.pad(x, ((0, NPAD - N), (0, 0)))

    deg = _sc_degree(src3, dst3)
    degT = deg.reshape(4, NPAD).T
    ns, nd, hs = _tc_prep(degT, x_pad, W1)

    agg = _sc_segsum(hs, src16, dst16)
    hs = _tc_layer(agg, nd, ns, b1.reshape(1, H), W2)
    agg = _sc_segsum(hs, src16, dst16)
    hs = _tc_layer(agg, nd, ns, b2.reshape(1, H), W3)
    agg = _sc_segsum(hs, src16, dst16)
    hs = _tc_layer(agg, nd, ns, b3.reshape(1, H), W4)
    agg = _sc_segsum(hs, src16, dst16)

    p1, p2 = _tc_final(agg, nd, b4.reshape(1, H),
                       Wc[:H], Wc[H:], bc.reshape(1, C))
    g1, g2 = _sc_pred(p1, p2, src3, dst3)
    out = _tc_sigmoid(g1.reshape(_E2, 128), g2.reshape(_E2, 128))
    return out.reshape(E, C)


# speedup vs baseline: 9.0412x; 1.3583x over previous
import functools

import jax
import jax.numpy as jnp
from jax import lax
from jax.experimental import pallas as pl
from jax.experimental.pallas import tpu as pltpu
from jax.experimental.pallas import tpu_sc as plsc

N = 10000
E = 320000
D = 128
H = 128
C = 16

NC = 2
NS = 16
NW = NC * NS
NPAD = 10240
ROWS_PT = NPAD // NS
EPW = E // NW
CH = 80
NCH = EPW // CH
HH = H // 2
EPT = E // NS
NCH2 = EPT // CH

_MESH = plsc.VectorSubcoreMesh(
    core_axis_name="c", subcore_axis_name="s", num_cores=NC, num_subcores=NS)
_SC_PARAMS = pltpu.CompilerParams(use_tc_tiling_on_sc=False)


def _wid():
    return lax.axis_index("s") * NC + lax.axis_index("c")


@functools.partial(
    pl.kernel,
    out_type=jax.ShapeDtypeStruct((NC, 2, NPAD), jnp.float32),
    mesh=_MESH,
    compiler_params=_SC_PARAMS,
    scratch_types=[
        pltpu.VMEM((NCH, CH), jnp.int32),
        pltpu.VMEM((NCH, CH), jnp.int32),
        pltpu.VMEM((CH,), jnp.float32),
        pltpu.VMEM((ROWS_PT,), jnp.float32),
        pltpu.VMEM_SHARED((NPAD,), jnp.float32),
        pltpu.SemaphoreType.DMA,
    ],
)
def _sc_degree(src_hbm, dst_hbm, out_hbm, sidx, didx, ones_v, zbuf,
               deg_sh, sem0):
    c = lax.axis_index("c")
    s = lax.axis_index("s")
    w = _wid()
    base = s * ROWS_PT
    one = jnp.full((16,), 1.0, jnp.float32)
    zero = jnp.zeros((16,), jnp.float32)

    def fill(t, _):
        ones_v[pl.ds(t * 16, 16)] = one
        return 0
    lax.fori_loop(0, CH // 16, fill, 0)

    def zfill(t, _):
        zbuf[pl.ds(t * 16, 16)] = zero
        return 0
    lax.fori_loop(0, ROWS_PT // 16, zfill, 0)

    pltpu.sync_copy(src_hbm.at[w], sidx)
    pltpu.sync_copy(dst_hbm.at[w], didx)

    DEPTH = 8

    def histogram(idx, which):
        pltpu.sync_copy(zbuf, deg_sh.at[pl.ds(base, ROWS_PT)])
        plsc.subcore_barrier()

        def fire(j, _):
            @pl.when(j >= DEPTH)
            def _():
                pltpu.make_async_copy(ones_v, deg_sh.at[idx.at[j - DEPTH]],
                                      sem0).wait()
            pltpu.async_copy(ones_v, deg_sh.at[idx.at[j]], sem0, add=True)
            return 0
        lax.fori_loop(0, NCH, fire, 0)

        def drain(j, _):
            pltpu.make_async_copy(ones_v, deg_sh.at[idx.at[j]], sem0).wait()
            return 0
        lax.fori_loop(NCH - DEPTH, NCH, drain, 0)
        plsc.subcore_barrier()
        pltpu.sync_copy(deg_sh.at[pl.ds(base, ROWS_PT)],
                        out_hbm.at[c, which, pl.ds(base, ROWS_PT)])
        plsc.subcore_barrier()

    histogram(sidx, 0)
    histogram(didx, 1)


_NB = 5
_LAG = 2

assert NCH2 % _NB == 0

@functools.partial(
    pl.kernel,
    out_type=jax.ShapeDtypeStruct((NC, NPAD, HH), jnp.float32),
    mesh=_MESH,
    compiler_params=_SC_PARAMS,
    scratch_types=(
        [pltpu.VMEM((NCH2, CH), jnp.int32),
         pltpu.VMEM((NCH2, CH), jnp.int32)]
        + [pltpu.VMEM((CH, HH), jnp.float32)] * _NB
        + [pltpu.VMEM_SHARED((NPAD, HH), jnp.float32)]
        + [pltpu.SemaphoreType.DMA] * (2 * _NB)
    ),
)
def _sc_segsum(hs_hbm, src_hbm, dst_hbm, out_hbm, sidx, didx, *rest):
    rows = rest[:_NB]
    acc_sh = rest[_NB]
    gsems = rest[_NB + 1:2 * _NB + 1]
    ssems = rest[2 * _NB + 1:]
    c = lax.axis_index("c")
    s = lax.axis_index("s")
    base = s * ROWS_PT
    zero = jnp.zeros((16,), jnp.float32)
    hs_half = hs_hbm.at[c]

    def zstore(t, _):
        rows[0][t // (HH // 16), pl.ds((t % (HH // 16)) * 16, 16)] = zero
        return 0
    lax.fori_loop(0, CH * (HH // 16), zstore, 0)

    def zcopy(k, _):
        pltpu.sync_copy(rows[0], acc_sh.at[pl.ds(base + k * CH, CH)])
        return 0
    lax.fori_loop(0, ROWS_PT // CH, zcopy, 0)

    pltpu.sync_copy(src_hbm.at[s], sidx)
    pltpu.sync_copy(dst_hbm.at[s], didx)
    plsc.subcore_barrier()

    def gath(j, k):
        return pltpu.make_async_copy(hs_half.at[sidx.at[j]], rows[k],
                                     gsems[k])

    def scat(j, k):
        return pltpu.make_async_copy(rows[k], acc_sh.at[didx.at[j]],
                                     ssems[k])

    for k in range(_NB):
        gath(k, k).start()

    def body(jj, _):
        j0 = jj * _NB
        for k in range(_NB):
            j = j0 + k
            gath(j, k).wait()
            pltpu.async_copy(rows[k], acc_sh.at[didx.at[j]], ssems[k],
                             add=True)
            jl = j - _LAG
            kl = (k + _NB - _LAG) % _NB

            @pl.when(jl >= 0)
            def _(jl=jl, kl=kl):
                scat(jl, kl).wait()

                @pl.when(jl + _NB < NCH2)
                def _(jl=jl, kl=kl):
                    gath(jl + _NB, kl).start()
        return 0
    lax.fori_loop(0, NCH2 // _NB, body, 0)

    for i in range(_LAG):
        j = NCH2 - _LAG + i
        scat(j, j % _NB).wait()
    plsc.subcore_barrier()

    pltpu.sync_copy(acc_sh.at[pl.ds(base, ROWS_PT)],
                    out_hbm.at[c, pl.ds(base, ROWS_PT)])


@functools.partial(
    pl.kernel,
    out_type=(jax.ShapeDtypeStruct((E, C), jnp.float32),
              jax.ShapeDtypeStruct((E, C), jnp.float32)),
    mesh=_MESH,
    compiler_params=_SC_PARAMS,
    scratch_types=[
        pltpu.VMEM((NCH, CH), jnp.int32),
        pltpu.VMEM((NCH, CH), jnp.int32),
        pltpu.VMEM((CH, C), jnp.float32),
        pltpu.VMEM((CH, C), jnp.float32),
        pltpu.VMEM((CH, C), jnp.float32),
        pltpu.VMEM((CH, C), jnp.float32),
        pltpu.SemaphoreType.DMA,
        pltpu.SemaphoreType.DMA,
        pltpu.SemaphoreType.DMA,
        pltpu.SemaphoreType.DMA,
    ],
)
def _sc_pred(p1_hbm, p2_hbm, src_hbm, dst_hbm, g1_hbm, g2_hbm,
             sidx, didx, r1a, r2a, r1b, r2b, gsem0, gsem1, wsem0, wsem1):
    w = _wid()
    ebase = w * EPW

    pltpu.sync_copy(src_hbm.at[w], sidx)
    pltpu.sync_copy(dst_hbm.at[w], didx)

    pltpu.async_copy(p1_hbm.at[sidx.at[0]], r1a, gsem0)
    pltpu.async_copy(p2_hbm.at[didx.at[0]], r2a, gsem0)
    pltpu.async_copy(p1_hbm.at[sidx.at[1]], r1b, gsem1)
    pltpu.async_copy(p2_hbm.at[didx.at[1]], r2b, gsem1)

    def body(jj, _):
        a = 2 * jj
        b = a + 1
        pltpu.make_async_copy(p1_hbm.at[sidx.at[a]], r1a, gsem0).wait()
        pltpu.make_async_copy(p2_hbm.at[didx.at[a]], r2a, gsem0).wait()
        pltpu.async_copy(r1a, g1_hbm.at[pl.ds(ebase + a * CH, CH)], wsem0)
        pltpu.async_copy(r2a, g2_hbm.at[pl.ds(ebase + a * CH, CH)], wsem0)
        pltpu.make_async_copy(p1_hbm.at[sidx.at[b]], r1b, gsem1).wait()
        pltpu.make_async_copy(p2_hbm.at[didx.at[b]], r2b, gsem1).wait()
        pltpu.async_copy(r1b, g1_hbm.at[pl.ds(ebase + b * CH, CH)], wsem1)
        pltpu.async_copy(r2b, g2_hbm.at[pl.ds(ebase + b * CH, CH)], wsem1)
        pltpu.make_async_copy(r1a, g1_hbm.at[pl.ds(ebase + a * CH, CH)],
                              wsem0).wait()
        pltpu.make_async_copy(r2a, g2_hbm.at[pl.ds(ebase + a * CH, CH)],
                              wsem0).wait()

        @pl.when(a + 2 < NCH)
        def _():
            pltpu.async_copy(p1_hbm.at[sidx.at[a + 2]], r1a, gsem0)
            pltpu.async_copy(p2_hbm.at[didx.at[a + 2]], r2a, gsem0)
        pltpu.make_async_copy(r1b, g1_hbm.at[pl.ds(ebase + b * CH, CH)],
                              wsem1).wait()
        pltpu.make_async_copy(r2b, g2_hbm.at[pl.ds(ebase + b * CH, CH)],
                              wsem1).wait()

        @pl.when(b + 2 < NCH)
        def _():
            pltpu.async_copy(p1_hbm.at[sidx.at[b + 2]], r1b, gsem1)
            pltpu.async_copy(p2_hbm.at[didx.at[b + 2]], r2b, gsem1)
        return 0
    lax.fori_loop(0, NCH // 2, body, 0)

    j = NCH - 1
    pltpu.make_async_copy(p1_hbm.at[sidx.at[j]], r1a, gsem0).wait()
    pltpu.make_async_copy(p2_hbm.at[didx.at[j]], r2a, gsem0).wait()
    pltpu.sync_copy(r1a, g1_hbm.at[pl.ds(ebase + j * CH, CH)])
    pltpu.sync_copy(r2a, g2_hbm.at[pl.ds(ebase + j * CH, CH)])


def _tc_prep(degT, x, W1):
    def body(d_ref, x_ref, w_ref, ns_ref, nd_ref, hs_ref):
        d = d_ref[...]
        out_deg = d[:, 0:1] + d[:, 2:3]
        in_deg = d[:, 1:2] + d[:, 3:4]
        ns = lax.rsqrt(jnp.maximum(out_deg, 1.0))
        nd = lax.rsqrt(jnp.maximum(in_deg, 1.0))
        ns_ref[...] = ns
        nd_ref[...] = nd
        r = jnp.dot(x_ref[...] * ns, w_ref[...],
                    preferred_element_type=jnp.float32)
        hs_ref[0, :, :] = r[:, :HH]
        hs_ref[1, :, :] = r[:, HH:]
    return pl.pallas_call(
        body,
        out_shape=(jax.ShapeDtypeStruct((NPAD, 1), jnp.float32),
                   jax.ShapeDtypeStruct((NPAD, 1), jnp.float32),
                   jax.ShapeDtypeStruct((NC, NPAD, HH), jnp.float32)),
    )(degT, x, W1)


def _tc_layer(agg, nd, ns, b, Wn):
    def body(a_ref, nd_ref, ns_ref, b_ref, w_ref, out_ref):
        nd_v = nd_ref[...]
        h_lo = jnp.maximum(a_ref[0, :, :] * nd_v + b_ref[:, :HH], 0.0)
        h_hi = jnp.maximum(a_ref[1, :, :] * nd_v + b_ref[:, HH:], 0.0)
        ns_v = ns_ref[...]
        r = (jnp.dot(h_lo * ns_v, w_ref[:HH, :],
                     preferred_element_type=jnp.float32)
             + jnp.dot(h_hi * ns_v, w_ref[HH:, :],
                       preferred_element_type=jnp.float32))
        out_ref[0, :, :] = r[:, :HH]
        out_ref[1, :, :] = r[:, HH:]
    return pl.pallas_call(
        body,
        out_shape=jax.ShapeDtypeStruct((NC, NPAD, HH), jnp.float32),
    )(agg, nd, ns, b, Wn)


def _tc_final(agg, nd, b, WcA, WcB, bc):
    def body(a_ref, nd_ref, b_ref, wa_ref, wb_ref, bc_ref, p1_ref, p2_ref):
        nd_v = nd_ref[...]
        h_lo = jnp.maximum(a_ref[0, :, :] * nd_v + b_ref[:, :HH], 0.0)
        h_hi = jnp.maximum(a_ref[1, :, :] * nd_v + b_ref[:, HH:], 0.0)
        p1_ref[...] = (jnp.dot(h_lo, wa_ref[:HH, :],
                               preferred_element_type=jnp.float32)
                       + jnp.dot(h_hi, wa_ref[HH:, :],
                                 preferred_element_type=jnp.float32)
                       + bc_ref[...])
        p2_ref[...] = (jnp.dot(h_lo, wb_ref[:HH, :],
                               preferred_element_type=jnp.float32)
                       + jnp.dot(h_hi, wb_ref[HH:, :],
                                 preferred_element_type=jnp.float32))
    return pl.pallas_call(
        body,
        out_shape=(jax.ShapeDtypeStruct((NPAD, C), jnp.float32),
                   jax.ShapeDtypeStruct((NPAD, C), jnp.float32)),
    )(agg, nd, b, WcA, WcB, bc)


_E2 = E * C // 128
_EB = _E2 // 8


def _tc_sigmoid(g1, g2):
    def body(g1_ref, g2_ref, out_ref):
        z = g1_ref[...] + g2_ref[...]
        out_ref[...] = 1.0 / (1.0 + jnp.exp(-z))
    return pl.pallas_call(
        body,
        grid=(8,),
        in_specs=[pl.BlockSpec((_EB, 128), lambda i: (i, 0)),
                  pl.BlockSpec((_EB, 128), lambda i: (i, 0))],
        out_specs=pl.BlockSpec((_EB, 128), lambda i: (i, 0)),
        out_shape=jax.ShapeDtypeStruct((_E2, 128), jnp.float32),
    )(g1, g2)


def kernel(x, edge_index, W1, b1, W2, b2, W3, b3, W4, b4, Wc, bc):
    src3 = edge_index[0].reshape(NW, NCH, CH)
    dst3 = edge_index[1].reshape(NW, NCH, CH)
    src16 = edge_index[0].reshape(NS, NCH2, CH)
    dst16 = edge_index[1].reshape(NS, NCH2, CH)
    x_pad = jnp.pad(x, ((0, NPAD - N), (0, 0)))

    deg = _sc_degree(src3, dst3)
    degT = deg.reshape(4, NPAD).T
    ns, nd, hs = _tc_prep(degT, x_pad, W1)

    agg = _sc_segsum(hs, src16, dst16)
    hs = _tc_layer(agg, nd, ns, b1.reshape(1, H), W2)
    agg = _sc_segsum(hs, src16, dst16)
    hs = _tc_layer(agg, nd, ns, b2.reshape(1, H), W3)
    agg = _sc_segsum(hs, src16, dst16)
    hs = _tc_layer(agg, nd, ns, b3.reshape(1, H), W4)
    agg = _sc_segsum(hs, src16, dst16)

    p1, p2 = _tc_final(agg, nd, b4.reshape(1, H),
                       Wc[:H], Wc[H:], bc.reshape(1, C))
    g1, g2 = _sc_pred(p1, p2, src3, dst3)
    out = _tc_sigmoid(g1.reshape(_E2, 128), g2.reshape(_E2, 128))
    return out.reshape(E, C)
